# Initial kernel scaffold; baseline (speedup 1.0000x reference)
import jax
import jax.numpy as jnp
from jax.experimental import pallas as pl


def kernel(img, rois):
    raise NotImplementedError("write your pallas kernel here")


# speedup vs baseline: 3.5936x; 3.5936x over previous
import jax
import jax.numpy as jnp
from jax import lax
from jax.experimental import pallas as pl
from jax.experimental.pallas import tpu as pltpu
from jax.experimental.pallas import tpu_sc as plsc

_POOL = 7
_NUM_ROIS = 300
_H = 64
_W = 64
_C = 512
_RPR = _POOL * _POOL
_NROWS = _NUM_ROIS * _RPR
_CHUNK = 16
_NCHUNKS = -(-_NROWS // _CHUNK)
_TAIL = _NROWS - (_NCHUNKS - 1) * _CHUNK

_info = plsc.get_sparse_core_info()
_NC = _info.num_cores
_NS = _info.num_subcores
_NW = _NC * _NS
_CV = _C // 16
_CPW = -(-_NCHUNKS // _NW)


def _body(img_hbm, rois_hbm, out_hbm, rois_v, idx_v, oidx_v, wrow, rows_v,
          out_v, sem):
  wid = lax.axis_index("s") * _NC + lax.axis_index("c")

  pltpu.sync_copy(rois_hbm, rois_v)

  lanes = lax.iota(jnp.int32, 16)

  def full16(v):
    return jnp.full((16,), v, jnp.int32)

  def do_chunk(c):
    m0 = jnp.where(c == _NCHUNKS - 1, _NROWS - _CHUNK, c * _CHUNK)
    m = full16(m0) + lanes
    r = lax.div(m, _RPR)
    k = m - r * _RPR
    py = lax.div(k, _POOL)
    px = k - py * _POOL

    xv = plsc.load_gather(rois_v, [4 * r])
    yv = plsc.load_gather(rois_v, [4 * r + 1])
    wv = plsc.load_gather(rois_v, [4 * r + 2])
    hv = plsc.load_gather(rois_v, [4 * r + 3])

    ys = py.astype(jnp.float32) * (hv.astype(jnp.float32) / float(_POOL))
    y0 = jnp.minimum(ys.astype(jnp.int32), hv - 1)
    y1 = jnp.minimum(y0 + 1, hv - 1)
    wy = ys - y0.astype(jnp.float32)

    xs = px.astype(jnp.float32) * (wv.astype(jnp.float32) / float(_POOL))
    x0 = jnp.minimum(xs.astype(jnp.int32), wv - 1)
    x1 = jnp.minimum(x0 + 1, wv - 1)
    wx = xs - x0.astype(jnp.float32)

    ro0 = (yv + y0) * _W
    ro1 = (yv + y1) * _W
    ca = xv + x0
    cb = xv + x1

    idx_v[pl.ds(0, 16)] = ro0 + ca
    idx_v[pl.ds(16, 16)] = ro0 + cb
    idx_v[pl.ds(32, 16)] = ro1 + ca
    idx_v[pl.ds(48, 16)] = ro1 + cb
    omwy = 1.0 - wy
    omwx = 1.0 - wx
    wrow[pl.ds(0, 16)] = omwy * omwx
    wrow[pl.ds(16, 16)] = omwy * wx
    wrow[pl.ds(32, 16)] = wy * omwx
    wrow[pl.ds(48, 16)] = wy * wx

    pltpu.async_copy(img_hbm.at[idx_v], rows_v, sem).wait()

    def do_row(j, _):
      wa = plsc.load_gather(wrow, [full16(j)])
      wb = plsc.load_gather(wrow, [full16(j + 16)])
      wc = plsc.load_gather(wrow, [full16(j + 32)])
      wd = plsc.load_gather(wrow, [full16(j + 48)])
      for v in range(_CV):
        sl = pl.ds(v * 16, 16)
        acc = (rows_v[j, sl] * wa + rows_v[j + 16, sl] * wb
               + rows_v[j + 32, sl] * wc + rows_v[j + 48, sl] * wd)
        out_v[j, sl] = acc
      return 0

    lax.fori_loop(0, _CHUNK, do_row, 0)

    @pl.when(c < _NCHUNKS - 1)
    def _():
      pltpu.sync_copy(out_v, out_hbm.at[pl.ds(c * _CHUNK, _CHUNK)])

    @pl.when(c == _NCHUNKS - 1)
    def _():
      oidx_v[...] = full16(_NROWS - _CHUNK) + lanes
      pltpu.async_copy(out_v, out_hbm.at[oidx_v], sem).wait()

  def do_j(j, _):
    c = wid + j * _NW

    @pl.when(c < _NCHUNKS)
    def _():
      do_chunk(c)

    return 0

  lax.fori_loop(0, _CPW, do_j, 0)


@jax.jit
def kernel(img, rois):
  img2 = img.reshape(_H * _W, _C)
  rflat = rois.reshape(-1).astype(jnp.int32)
  mesh = plsc.VectorSubcoreMesh(core_axis_name="c", subcore_axis_name="s")
  out = pl.kernel(
      _body,
      mesh=mesh,
      compiler_params=pltpu.CompilerParams(needs_layout_passes=False),
      out_type=jax.ShapeDtypeStruct((_NROWS, _C), jnp.float32),
      scratch_types=[
          pltpu.VMEM((_NUM_ROIS * 4,), jnp.int32),
          pltpu.VMEM((64,), jnp.int32),
          pltpu.VMEM((16,), jnp.int32),
          pltpu.VMEM((64,), jnp.float32),
          pltpu.VMEM((64, _C), jnp.float32),
          pltpu.VMEM((_CHUNK, _C), jnp.float32),
          pltpu.SemaphoreType.DMA,
      ],
  )(img2, rflat)
  return out.reshape(1, _NUM_ROIS, _POOL, _POOL, _C)


# speedup vs baseline: 4.5918x; 1.2778x over previous
import jax
import jax.numpy as jnp
from jax import lax
from jax.experimental import pallas as pl
from jax.experimental.pallas import tpu as pltpu
from jax.experimental.pallas import tpu_sc as plsc

_POOL = 7
_NUM_ROIS = 300
_H = 64
_W = 64
_C = 512
_RPR = _POOL * _POOL
_NROWS = _NUM_ROIS * _RPR
_CHUNK = 16
_NCHUNKS = -(-_NROWS // _CHUNK)

_info = plsc.get_sparse_core_info()
_NC = _info.num_cores
_NS = _info.num_subcores
_NW = _NC * _NS
_CV = _C // 16
_CPW = -(-_NCHUNKS // _NW)
_NPAIR = (_CPW + 2) // 2


def _body(img_hbm, rois_hbm, out_hbm, rois_v,
          idx0, idx1, oidx_v, wrow0, wrow1, rows0, rows1, out0, out1,
          gsem0, gsem1, wsem0, wsem1):
  idx = (idx0, idx1)
  wrow = (wrow0, wrow1)
  rows = (rows0, rows1)
  outv = (out0, out1)
  gsem = (gsem0, gsem1)
  wsem = (wsem0, wsem1)

  wid = lax.axis_index("s") * _NC + lax.axis_index("c")

  pltpu.sync_copy(rois_hbm, rois_v)

  lanes = lax.iota(jnp.int32, 16)

  def full16(v):
    return jnp.full((16,), v, jnp.int32)

  def chunk_of(j):
    return wid + j * _NW

  def fire(c, b):
    m0 = jnp.where(c == _NCHUNKS - 1, _NROWS - _CHUNK, c * _CHUNK)
    m = full16(m0) + lanes
    r = lax.div(m, _RPR)
    k = m - r * _RPR
    py = lax.div(k, _POOL)
    px = k - py * _POOL

    xv = plsc.load_gather(rois_v, [4 * r])
    yv = plsc.load_gather(rois_v, [4 * r + 1])
    wv = plsc.load_gather(rois_v, [4 * r + 2])
    hv = plsc.load_gather(rois_v, [4 * r + 3])

    ys = py.astype(jnp.float32) * (hv.astype(jnp.float32) / float(_POOL))
    y0 = jnp.minimum(ys.astype(jnp.int32), hv - 1)
    y1 = jnp.minimum(y0 + 1, hv - 1)
    wy = ys - y0.astype(jnp.float32)

    xs = px.astype(jnp.float32) * (wv.astype(jnp.float32) / float(_POOL))
    x0 = jnp.minimum(xs.astype(jnp.int32), wv - 1)
    x1 = jnp.minimum(x0 + 1, wv - 1)
    wx = xs - x0.astype(jnp.float32)

    ro0 = (yv + y0) * _W
    ro1 = (yv + y1) * _W
    ca = xv + x0
    cb = xv + x1

    idx[b][pl.ds(0, 16)] = ro0 + ca
    idx[b][pl.ds(16, 16)] = ro0 + cb
    idx[b][pl.ds(32, 16)] = ro1 + ca
    idx[b][pl.ds(48, 16)] = ro1 + cb
    omwy = 1.0 - wy
    omwx = 1.0 - wx
    wrow[b][pl.ds(0, 16)] = omwy * omwx
    wrow[b][pl.ds(16, 16)] = omwy * wx
    wrow[b][pl.ds(32, 16)] = wy * omwx
    wrow[b][pl.ds(48, 16)] = wy * wx

    pltpu.async_copy(img_hbm.at[idx[b]], rows[b], gsem[b])

  def blend(b):
    def do_row(j, _):
      wa = plsc.load_gather(wrow[b], [full16(j)])
      wb = plsc.load_gather(wrow[b], [full16(j + 16)])
      wc = plsc.load_gather(wrow[b], [full16(j + 32)])
      wd = plsc.load_gather(wrow[b], [full16(j + 48)])
      rv = rows[b]
      ov = outv[b]
      for v in range(_CV):
        sl = pl.ds(v * 16, 16)
        acc = (rv[j, sl] * wa + rv[j + 16, sl] * wb
               + rv[j + 32, sl] * wc + rv[j + 48, sl] * wd)
        ov[j, sl] = acc
      return 0

    lax.fori_loop(0, _CHUNK, do_row, 0)

  def write(c, b):
    @pl.when(c < _NCHUNKS - 1)
    def _():
      pltpu.async_copy(outv[b], out_hbm.at[pl.ds(c * _CHUNK, _CHUNK)], wsem[b])

    @pl.when(c == _NCHUNKS - 1)
    def _():
      oidx_v[...] = full16(_NROWS - _CHUNK) + lanes
      pltpu.async_copy(outv[b], out_hbm.at[oidx_v], wsem[b])

  def wait_gather(b):
    pltpu.make_async_copy(img_hbm.at[idx[b]], rows[b], gsem[b]).wait()

  def wait_write(b):
    pltpu.make_async_copy(outv[b], out_hbm.at[pl.ds(0, _CHUNK)],
                          wsem[b]).wait()

  @pl.when(chunk_of(0) < _NCHUNKS)
  def _():
    fire(chunk_of(0), 0)

  def pair_body(t, _):
    for b in (0, 1):
      j = 2 * t + b
      c = chunk_of(j)
      cn = chunk_of(j + 1)

      @pl.when(cn < _NCHUNKS)
      def _():
        fire(cn, 1 - b)

      @pl.when(c < _NCHUNKS)
      def _():
        wait_gather(b)
        @pl.when(j >= 2)
        def _():
          wait_write(b)

        blend(b)
        write(c, b)

    return 0

  lax.fori_loop(0, _NPAIR, pair_body, 0)

  wait_write(0)
  wait_write(1)


@jax.jit
def kernel(img, rois):
  img2 = img.reshape(_H * _W, _C)
  rflat = rois.reshape(-1).astype(jnp.int32)
  mesh = plsc.VectorSubcoreMesh(core_axis_name="c", subcore_axis_name="s")
  out = pl.kernel(
      _body,
      mesh=mesh,
      compiler_params=pltpu.CompilerParams(needs_layout_passes=False),
      out_type=jax.ShapeDtypeStruct((_NROWS, _C), jnp.float32),
      scratch_types=[
          pltpu.VMEM((_NUM_ROIS * 4,), jnp.int32),
          pltpu.VMEM((64,), jnp.int32),
          pltpu.VMEM((64,), jnp.int32),
          pltpu.VMEM((16,), jnp.int32),
          pltpu.VMEM((64,), jnp.float32),
          pltpu.VMEM((64,), jnp.float32),
          pltpu.VMEM((64, _C), jnp.float32),
          pltpu.VMEM((64, _C), jnp.float32),
          pltpu.VMEM((_CHUNK, _C), jnp.float32),
          pltpu.VMEM((_CHUNK, _C), jnp.float32),
          pltpu.SemaphoreType.DMA,
          pltpu.SemaphoreType.DMA,
          pltpu.SemaphoreType.DMA,
          pltpu.SemaphoreType.DMA,
      ],
  )(img2, rflat)
  return out.reshape(1, _NUM_ROIS, _POOL, _POOL, _C)


# speedup vs baseline: 4.8897x; 1.0649x over previous
import jax
import jax.numpy as jnp
from jax import lax
from jax.experimental import pallas as pl
from jax.experimental.pallas import tpu as pltpu
from jax.experimental.pallas import tpu_sc as plsc

_POOL = 7
_NUM_ROIS = 300
_H = 64
_W = 64
_C = 512
_NUNITS = _NUM_ROIS * _POOL

_info = plsc.get_sparse_core_info()
_NC = _info.num_cores
_NS = _info.num_subcores
_NW = _NC * _NS
_CV = _C // 16
_UPW = -(-_NUNITS // _NW)
_NPAIR = (_UPW + 2) // 2


def _body(img_hbm, rois_hbm, out_hbm, rois_v,
          idx0, idx1, wrow0, wrow1, rows0, rows1, out0, out1,
          gsem0, gsem1, wsem0, wsem1):
  idx = (idx0, idx1)
  wrow = (wrow0, wrow1)
  rows = (rows0, rows1)
  outv = (out0, out1)
  gsem = (gsem0, gsem1)
  wsem = (wsem0, wsem1)

  wid = lax.axis_index("s") * _NC + lax.axis_index("c")

  pltpu.sync_copy(rois_hbm, rois_v)

  lanes = lax.iota(jnp.int32, 16)
  pxv = jnp.minimum(jnp.where(lanes < 7, lanes, lanes - 7), 6)
  grpb = lanes >= 7
  live = lanes < 14

  def full16(v):
    return jnp.full((16,), v, jnp.int32)

  def unit_of(j):
    return wid + j * _NW

  def fire(u, b):
    r = lax.div(u, _POOL)
    py = u - r * _POOL

    xv = plsc.load_gather(rois_v, [full16(4 * r)])
    yv = plsc.load_gather(rois_v, [full16(4 * r + 1)])
    wv = plsc.load_gather(rois_v, [full16(4 * r + 2)])
    hv = plsc.load_gather(rois_v, [full16(4 * r + 3)])

    ys = full16(py).astype(jnp.float32) * (hv.astype(jnp.float32)
                                           / float(_POOL))
    y0 = jnp.minimum(ys.astype(jnp.int32), hv - 1)
    y1 = jnp.minimum(y0 + 1, hv - 1)
    wy = ys - y0.astype(jnp.float32)

    xs = pxv.astype(jnp.float32) * (wv.astype(jnp.float32) / float(_POOL))
    x0 = jnp.minimum(xs.astype(jnp.int32), wv - 1)
    x1 = jnp.minimum(x0 + 1, wv - 1)
    wx = xs - x0.astype(jnp.float32)

    col = xv + jnp.where(grpb, x1, x0)
    wcol = jnp.where(live, jnp.where(grpb, wx, 1.0 - wx), 0.0)

    idx[b][pl.ds(0, 16)] = (yv + y0) * _W + col
    idx[b][pl.ds(16, 16)] = (yv + y1) * _W + col
    wrow[b][pl.ds(0, 16)] = (1.0 - wy) * wcol
    wrow[b][pl.ds(16, 16)] = wy * wcol

    pltpu.async_copy(img_hbm.at[idx[b]], rows[b], gsem[b])

  def blend(b):
    rv = rows[b]
    ov = outv[b]

    def do_px(px, _):
      wa = plsc.load_gather(wrow[b], [full16(px)])
      wb = plsc.load_gather(wrow[b], [full16(px + 7)])
      wc = plsc.load_gather(wrow[b], [full16(px + 16)])
      wd = plsc.load_gather(wrow[b], [full16(px + 23)])
      for v in range(_CV):
        sl = pl.ds(v * 16, 16)
        acc = (rv[px, sl] * wa + rv[px + 7, sl] * wb
               + rv[px + 16, sl] * wc + rv[px + 23, sl] * wd)
        ov[px, sl] = acc
      return 0

    lax.fori_loop(0, _POOL, do_px, 0)

  def write(u, b):
    pltpu.async_copy(outv[b], out_hbm.at[u], wsem[b])

  def wait_gather(b):
    pltpu.make_async_copy(img_hbm.at[idx[b]], rows[b], gsem[b]).wait()

  def wait_write(b):
    pltpu.make_async_copy(outv[b], out_hbm.at[0], wsem[b]).wait()

  fire(unit_of(0), 0)

  def pair_body(t, _):
    for b in (0, 1):
      j = 2 * t + b
      u = unit_of(j)
      un = unit_of(j + 1)

      @pl.when(un < _NUNITS)
      def _():
        fire(un, 1 - b)

      @pl.when(u < _NUNITS)
      def _():
        wait_gather(b)
        @pl.when(j >= 2)
        def _():
          wait_write(b)

        blend(b)
        write(u, b)

    return 0

  lax.fori_loop(0, _NPAIR, pair_body, 0)

  wait_write(0)
  wait_write(1)


_G = 10


def _depad_body(in_ref, out_ref):
  x = in_ref[...].reshape(_G, _POOL, 8, _C)
  out_ref[0] = x[:, :, :_POOL, :]


@jax.jit
def kernel(img, rois):
  img2 = img.reshape(_H * _W, _C)
  rflat = rois.reshape(-1).astype(jnp.int32)
  mesh = plsc.VectorSubcoreMesh(core_axis_name="c", subcore_axis_name="s")
  staged = pl.kernel(
      _body,
      mesh=mesh,
      compiler_params=pltpu.CompilerParams(needs_layout_passes=False),
      out_type=jax.ShapeDtypeStruct((_NUNITS, 8, _C), jnp.float32),
      scratch_types=[
          pltpu.VMEM((_NUM_ROIS * 4,), jnp.int32),
          pltpu.VMEM((32,), jnp.int32),
          pltpu.VMEM((32,), jnp.int32),
          pltpu.VMEM((32,), jnp.float32),
          pltpu.VMEM((32,), jnp.float32),
          pltpu.VMEM((32, _C), jnp.float32),
          pltpu.VMEM((32, _C), jnp.float32),
          pltpu.VMEM((8, _C), jnp.float32),
          pltpu.VMEM((8, _C), jnp.float32),
          pltpu.SemaphoreType.DMA,
          pltpu.SemaphoreType.DMA,
          pltpu.SemaphoreType.DMA,
          pltpu.SemaphoreType.DMA,
      ],
  )(img2, rflat)

  out = pl.pallas_call(
      _depad_body,
      grid=(_NUM_ROIS // _G,),
      in_specs=[pl.BlockSpec((_G * _POOL, 8, _C), lambda i: (i, 0, 0))],
      out_specs=pl.BlockSpec((1, _G, _POOL, _POOL, _C),
                             lambda i: (0, i, 0, 0, 0)),
      out_shape=jax.ShapeDtypeStruct((1, _NUM_ROIS, _POOL, _POOL, _C),
                                     jnp.float32),
  )(staged)
  return out
